# baseline (speedup 1.0000x reference)
import functools

import jax
import jax.numpy as jnp
from jax import lax
from jax.experimental import pallas as pl
from jax.experimental.pallas import tpu as pltpu
from jax.experimental.pallas import tpu_sc as plsc

VOCAB_ = 1000000
EMBED_ = 64
B_ = 4096
T_ = 200
TP1_ = T_ + 1
NC_ = 2
NS_ = 16
NW_ = NC_ * NS_
BPW_ = B_ // NW_
NBUF_ = 4
LANES_ = 16
NEG_ = EMBED_ // LANES_
NBG_ = BPW_ // LANES_


def _sc_body(emb_hbm, pht_hbm, rtt_hbm, tp_hbm, pos_hbm, out_hbm,
             idx_v, rt_v, rows_v, pos_v, semi, semg, semo, semt):
    wid = lax.axis_index("s") * NC_ + lax.axis_index("c")
    b0 = wid * BPW_

    pltpu.sync_copy(pos_hbm, pos_v)

    def fire_prefetch(t, p):
        pltpu.async_copy(pht_hbm.at[t, pl.ds(b0, BPW_)], idx_v.at[p],
                         semi.at[p])
        pltpu.async_copy(rtt_hbm.at[t, pl.ds(b0, BPW_)], rt_v.at[p],
                         semi.at[p])

    def fire_gather(t, p):
        pltpu.make_async_copy(pht_hbm.at[0, pl.ds(0, BPW_)], idx_v.at[p],
                              semi.at[p]).wait()
        pltpu.make_async_copy(rtt_hbm.at[0, pl.ds(0, BPW_)], rt_v.at[p],
                              semi.at[p]).wait()

        if not (isinstance(t, int) and t < NBUF_):
            @pl.when(t >= NBUF_)
            def _():
                pltpu.make_async_copy(
                    rows_v.at[p], out_hbm.at[pl.ds(0, BPW_), 0, :],
                    semo.at[p]).wait()

        pltpu.async_copy(emb_hbm.at[idx_v.at[p]], rows_v.at[p], semg.at[p])

    def compute_and_write(t, q):
        pltpu.make_async_copy(emb_hbm.at[idx_v.at[q]], rows_v.at[q],
                              semg.at[q]).wait()

        pv = [pos_v[t, pl.ds(j * LANES_, LANES_)] for j in range(NEG_)]
        for bg in range(NBG_):
            svec = rt_v[q, pl.ds(bg * LANES_, LANES_)]
            ss = [svec[i] for i in range(LANES_)]
            for i0 in range(0, LANES_, 4):
                vals = []
                for i in range(i0, i0 + 4):
                    r = bg * LANES_ + i
                    for j in range(NEG_):
                        sl = pl.ds(j * LANES_, LANES_)
                        vals.append((i, r, sl, rows_v[q, r, sl] + pv[j]))
                for (i, r, sl, v) in vals:
                    rows_v[q, r, sl] = v * ss[i]

        pltpu.async_copy(rows_v.at[q], out_hbm.at[pl.ds(b0, BPW_), t, :],
                         semo.at[q])

    fire_prefetch(0, 0)
    fire_prefetch(1, 1)
    fire_gather(0, 0)

    @pl.loop(0, T_)
    def _step(t):
        @pl.when(t + 2 < T_)
        def _():
            fire_prefetch(t + 2, lax.rem(t + 2, NBUF_))

        @pl.when(t + 1 < T_)
        def _():
            fire_gather(t + 1, lax.rem(t + 1, NBUF_))

        compute_and_write(t, lax.rem(t, NBUF_))

    pltpu.sync_copy(tp_hbm.at[pl.ds(b0, BPW_)], idx_v.at[0])
    pltpu.make_async_copy(rows_v.at[0], out_hbm.at[pl.ds(0, BPW_), 0, :],
                          semo.at[0]).wait()
    pltpu.async_copy(emb_hbm.at[idx_v.at[0]], rows_v.at[0], semt).wait()
    pltpu.sync_copy(rows_v.at[0], out_hbm.at[pl.ds(b0, BPW_), T_, :])

    for q in range(1, NBUF_):
        pltpu.make_async_copy(rows_v.at[q], out_hbm.at[pl.ds(0, BPW_), 0, :],
                              semo.at[q]).wait()


@jax.jit
def _run_sc(emb_weights, pht, rtt, tp_flat, pos_weights):
    mesh = plsc.VectorSubcoreMesh(core_axis_name="c", subcore_axis_name="s")
    fn = pl.kernel(
        _sc_body,
        out_type=jax.ShapeDtypeStruct((B_, TP1_, EMBED_), jnp.float32),
        mesh=mesh,
        scratch_types=[
            pltpu.VMEM((NBUF_, BPW_), jnp.int32),
            pltpu.VMEM((NBUF_, BPW_), jnp.float32),
            pltpu.VMEM((NBUF_, BPW_, EMBED_), jnp.float32),
            pltpu.VMEM((T_, EMBED_), jnp.float32),
            pltpu.SemaphoreType.DMA((NBUF_,)),
            pltpu.SemaphoreType.DMA((NBUF_,)),
            pltpu.SemaphoreType.DMA((NBUF_,)),
            pltpu.SemaphoreType.DMA,
        ],
        compiler_params=pltpu.CompilerParams(use_tc_tiling_on_sc=False),
    )
    return fn(emb_weights, pht, rtt, tp_flat, pos_weights)


def kernel(user_id, product_history, target_product_id,
           product_history_ratings, emb_weights, pos_weights):
    del user_id
    pht = product_history.astype(jnp.int32).T
    rtt = product_history_ratings.T
    tp_flat = target_product_id.astype(jnp.int32).reshape(B_)
    return _run_sc(emb_weights, pht, rtt, tp_flat, pos_weights)
